# Initial kernel scaffold; baseline (speedup 1.0000x reference)
import jax
import jax.numpy as jnp
from jax.experimental import pallas as pl


def kernel(h, w1, b1, w2, b2, wg, bg):
    raise NotImplementedError("write your pallas kernel here")


# speedup vs baseline: 2.4463x; 2.4463x over previous
import functools
import math

import jax
import jax.numpy as jnp
from jax import lax
from jax.experimental import pallas as pl
from jax.experimental.pallas import tpu as pltpu
from jax.experimental.pallas import tpu_sc as plsc

D_MODEL = 1024
D_FF = 4096
NUM_EXPERTS = 8
TOP_K = 2
T = 2048
CAP = math.ceil(1.25 * (T * TOP_K / NUM_EXPERTS))
BLK = 256
NB = (TOP_K * T) // BLK + NUM_EXPERTS
NR = NB * BLK
NTRASH = 8

NC, NS = 2, 16
NW = NC * NS
TPW = T // NW
CHUNK = 32


def _router_body(x_ref, wg_ref, bg_ref,
                 ds0_ref, ds1_ref, dg0_ref, dg1_ref, g0_ref, g1_ref,
                 be_ref, bv_ref):
    x = x_ref[...]
    wg = wg_ref[...]
    logits = lax.dot_general(x, wg, (((1,), (1,)), ((), ())),
                             preferred_element_type=jnp.float32)
    logits = logits + bg_ref[...]
    m = jnp.max(logits, axis=1, keepdims=True)
    ex = jnp.exp(logits - m)
    p = ex / jnp.sum(ex, axis=1, keepdims=True)

    iota8 = lax.broadcasted_iota(jnp.int32, (T, NUM_EXPERTS), 1)
    p0 = jnp.max(p, axis=1, keepdims=True)
    e0 = jnp.min(jnp.where(p == p0, iota8, NUM_EXPERTS + 1), axis=1,
                 keepdims=True)
    pm = jnp.where(iota8 == e0, -jnp.inf, p)
    p1 = jnp.max(pm, axis=1, keepdims=True)
    e1 = jnp.min(jnp.where(pm == p1, iota8, NUM_EXPERTS + 1), axis=1,
                 keepdims=True)

    oh0 = (iota8 == e0).astype(jnp.float32)
    oh1 = (iota8 == e1).astype(jnp.float32)
    M = jnp.concatenate([oh0, oh1], axis=1)

    row_i = lax.broadcasted_iota(jnp.int32, (T, T), 0)
    col_j = lax.broadcasted_iota(jnp.int32, (T, T), 1)
    tri = (col_j < row_i).astype(jnp.bfloat16)
    pos = lax.dot_general(tri, M.astype(jnp.bfloat16),
                          (((1,), (0,)), ((), ())),
                          preferred_element_type=jnp.float32)

    counts = jnp.sum(M, axis=0, keepdims=True)
    kept = jnp.minimum(counts, float(CAP))
    kept0 = kept[:, :NUM_EXPERTS]
    kept1 = kept[:, NUM_EXPERTS:]
    n_e = kept0 + kept1
    nblk = jnp.ceil(n_e / float(BLK))

    tri8 = (lax.broadcasted_iota(jnp.int32, (NUM_EXPERTS, NUM_EXPERTS), 0) <
            lax.broadcasted_iota(jnp.int32, (NUM_EXPERTS, NUM_EXPERTS), 1))
    offb = lax.dot_general(nblk, tri8.astype(jnp.float32),
                           (((1,), (0,)), ((), ())),
                           preferred_element_type=jnp.float32)
    off = offb * float(BLK)
    cumb = offb + nblk

    pos0 = pos[:, :NUM_EXPERTS]
    pos1 = pos[:, NUM_EXPERTS:]
    pos_t0 = jnp.sum(oh0 * pos0, axis=1, keepdims=True)
    pos_t1 = jnp.sum(oh1 * pos1, axis=1, keepdims=True)
    off_t0 = jnp.sum(oh0 * off, axis=1, keepdims=True)
    off_t1 = jnp.sum(oh1 * off, axis=1, keepdims=True)
    kept0_t1 = jnp.sum(oh1 * kept0, axis=1, keepdims=True)

    keep0 = pos_t0 < float(CAP)
    keep1 = pos_t1 < float(CAP)
    dest0 = (off_t0 + pos_t0).astype(jnp.int32)
    dest1 = (off_t1 + kept0_t1 + pos_t1).astype(jnp.int32)

    tok = lax.broadcasted_iota(jnp.int32, (T, 1), 0)
    trash = NR + (tok & (NTRASH - 1))
    ds0_ref[...] = jnp.where(keep0, dest0, trash)
    ds1_ref[...] = jnp.where(keep1, dest1, trash)
    dg0_ref[...] = jnp.where(keep0, dest0, 0)
    dg1_ref[...] = jnp.where(keep1, dest1, 0)
    g0_ref[...] = jnp.broadcast_to(jnp.where(keep0, p0, 0.0), (T, 16))
    g1_ref[...] = jnp.broadcast_to(jnp.where(keep1, p1, 0.0), (T, 16))

    bid = lax.broadcasted_iota(jnp.int32, (NB, NUM_EXPERTS), 0)
    eb = jnp.sum((bid >= cumb.astype(jnp.int32)).astype(jnp.int32),
                 axis=1, keepdims=True)
    total_b = jnp.sum(nblk, axis=1, keepdims=True).astype(jnp.int32)
    be_ref[...] = jnp.minimum(eb, NUM_EXPERTS - 1)
    bv_ref[...] = (lax.broadcasted_iota(jnp.int32, (NB, 1), 0)
                   < total_b).astype(jnp.int32)


def _router(x, wg, bg):
    outs = pl.pallas_call(
        _router_body,
        out_shape=[
            jax.ShapeDtypeStruct((T, 1), jnp.int32),
            jax.ShapeDtypeStruct((T, 1), jnp.int32),
            jax.ShapeDtypeStruct((T, 1), jnp.int32),
            jax.ShapeDtypeStruct((T, 1), jnp.int32),
            jax.ShapeDtypeStruct((T, 16), jnp.float32),
            jax.ShapeDtypeStruct((T, 16), jnp.float32),
            jax.ShapeDtypeStruct((NB, 1), jnp.int32),
            jax.ShapeDtypeStruct((NB, 1), jnp.int32),
        ],
    )(x, wg, bg.reshape(1, NUM_EXPERTS))
    return outs


def _dispatch_body(x_hbm, d0_hbm, d1_hbm, xs_hbm, xv, i0v, i1v, s0, s1):
    wid = lax.axis_index("s") * NC + lax.axis_index("c")
    base = wid * TPW
    pltpu.sync_copy(d0_hbm.at[pl.ds(base, TPW)], i0v)
    pltpu.sync_copy(d1_hbm.at[pl.ds(base, TPW)], i1v)
    pltpu.sync_copy(x_hbm.at[pl.ds(base, TPW)], xv)
    cp0 = pltpu.async_copy(xv, xs_hbm.at[i0v], s0)
    cp1 = pltpu.async_copy(xv, xs_hbm.at[i1v], s1)
    cp0.wait()
    cp1.wait()


@functools.cache
def _make_dispatch():
    return functools.partial(
        pl.kernel,
        out_type=jax.ShapeDtypeStruct((NR + NTRASH, D_MODEL), jnp.float32),
        mesh=plsc.VectorSubcoreMesh(core_axis_name="c", subcore_axis_name="s"),
        scratch_types=[
            pltpu.VMEM((TPW, D_MODEL), jnp.float32),
            pltpu.VMEM((TPW,), jnp.int32),
            pltpu.VMEM((TPW,), jnp.int32),
            pltpu.SemaphoreType.DMA,
            pltpu.SemaphoreType.DMA,
        ],
    )(_dispatch_body)


def _dispatch(x, ds0, ds1):
    return _make_dispatch()(x, ds0, ds1)


FBLK = 2048
NFF = D_FF // FBLK


def _ffn_body(be_ref, bv_ref, xs_ref, w1_ref, b1_ref, w2_ref, b2_ref, y_ref):
    b = pl.program_id(0)
    f = pl.program_id(1)

    @pl.when(bv_ref[b] != 0)
    def _():
        xb = xs_ref[...]
        h = lax.dot_general(xb, w1_ref[0], (((1,), (1,)), ((), ())),
                            preferred_element_type=jnp.float32)
        h = jnp.maximum(h + b1_ref[0], 0.0)
        part = lax.dot_general(h, w2_ref[0], (((1,), (1,)), ((), ())),
                               preferred_element_type=jnp.float32)

        @pl.when(f == 0)
        def _():
            y_ref[...] = part + b2_ref[0]

        @pl.when(f != 0)
        def _():
            y_ref[...] = y_ref[...] + part


def _ffn(be, bv, xs, w1, b1, w2, b2):
    grid_spec = pltpu.PrefetchScalarGridSpec(
        num_scalar_prefetch=2,
        grid=(NB, NFF),
        in_specs=[
            pl.BlockSpec((BLK, D_MODEL), lambda b, f, be, bv: (b, 0)),
            pl.BlockSpec((1, FBLK, D_MODEL),
                         lambda b, f, be, bv: (be[b], f, 0)),
            pl.BlockSpec((1, 1, FBLK), lambda b, f, be, bv: (be[b], 0, f)),
            pl.BlockSpec((1, D_MODEL, FBLK),
                         lambda b, f, be, bv: (be[b], 0, f)),
            pl.BlockSpec((1, 1, D_MODEL), lambda b, f, be, bv: (be[b], 0, 0)),
        ],
        out_specs=pl.BlockSpec((BLK, D_MODEL), lambda b, f, be, bv: (b, 0)),
    )
    return pl.pallas_call(
        _ffn_body,
        grid_spec=grid_spec,
        out_shape=jax.ShapeDtypeStruct((NR, D_MODEL), jnp.float32),
    )(be, bv, xs, w1,
      b1.reshape(NUM_EXPERTS, 1, D_FF),
      w2,
      b2.reshape(NUM_EXPERTS, 1, D_MODEL))


def _combine_body(y_hbm, i0_hbm, i1_hbm, g0_hbm, g1_hbm, out_hbm,
                  r0v, r1v, ov, i0v, i1v, g0v, g1v, s0, s1):
    wid = lax.axis_index("s") * NC + lax.axis_index("c")
    for chunk in range(TPW // CHUNK):
        base = wid * TPW + chunk * CHUNK
        pltpu.sync_copy(i0_hbm.at[pl.ds(base, CHUNK)], i0v)
        pltpu.sync_copy(i1_hbm.at[pl.ds(base, CHUNK)], i1v)
        pltpu.sync_copy(g0_hbm.at[pl.ds(base, CHUNK)], g0v)
        pltpu.sync_copy(g1_hbm.at[pl.ds(base, CHUNK)], g1v)
        cp0 = pltpu.async_copy(y_hbm.at[i0v], r0v, s0)
        cp1 = pltpu.async_copy(y_hbm.at[i1v], r1v, s1)
        cp0.wait()
        cp1.wait()

        def body(i, _):
            g0s = g0v[i, :]
            g1s = g1v[i, :]
            for j in range(D_MODEL // 16):
                sl = pl.ds(j * 16, 16)
                ov[i, sl] = g0s * r0v[i, sl] + g1s * r1v[i, sl]
            return 0

        lax.fori_loop(0, CHUNK, body, 0)
        pltpu.sync_copy(ov, out_hbm.at[pl.ds(base, CHUNK)])


@functools.cache
def _make_combine():
    return functools.partial(
        pl.kernel,
        out_type=jax.ShapeDtypeStruct((T, D_MODEL), jnp.float32),
        mesh=plsc.VectorSubcoreMesh(core_axis_name="c", subcore_axis_name="s"),
        scratch_types=[
            pltpu.VMEM((CHUNK, D_MODEL), jnp.float32),
            pltpu.VMEM((CHUNK, D_MODEL), jnp.float32),
            pltpu.VMEM((CHUNK, D_MODEL), jnp.float32),
            pltpu.VMEM((CHUNK,), jnp.int32),
            pltpu.VMEM((CHUNK,), jnp.int32),
            pltpu.VMEM((CHUNK, 16), jnp.float32),
            pltpu.VMEM((CHUNK, 16), jnp.float32),
            pltpu.SemaphoreType.DMA,
            pltpu.SemaphoreType.DMA,
        ],
    )(_combine_body)


def _combine(y, dg0, dg1, g0, g1):
    return _make_combine()(y, dg0, dg1, g0, g1)


def kernel(h, w1, b1, w2, b2, wg, bg):
    Bb, Ll, D = h.shape
    x = h.reshape(T, D_MODEL)
    ds0, ds1, dg0, dg1, g0, g1, be, bv = _router(x, wg, bg)
    xs = _dispatch(x, ds0.reshape(T), ds1.reshape(T))
    y = _ffn(be.reshape(NB), bv.reshape(NB), xs, w1, b1, w2, b2)
    out = _combine(y, dg0.reshape(T), dg1.reshape(T), g0, g1)
    return out.reshape(Bb, Ll, D)


# speedup vs baseline: 2.7153x; 1.1100x over previous
import functools
import math

import jax
import jax.numpy as jnp
from jax import lax
from jax.experimental import pallas as pl
from jax.experimental.pallas import tpu as pltpu
from jax.experimental.pallas import tpu_sc as plsc

D_MODEL = 1024
D_FF = 4096
NUM_EXPERTS = 8
TOP_K = 2
T = 2048
CAP = math.ceil(1.25 * (T * TOP_K / NUM_EXPERTS))
BLK = 256
NB = (TOP_K * T) // BLK + NUM_EXPERTS
NR = NB * BLK
NTRASH = 8

NC, NS = 2, 16
NW = NC * NS
TPW = T // NW
CHUNK = 16


def _router_body(x_ref, wg_ref, bg_ref,
                 ds0_ref, ds1_ref, dg0_ref, dg1_ref, dh0_ref, dh1_ref,
                 g0_ref, g1_ref, be_ref, bv_ref):
    x = x_ref[...]
    wg = wg_ref[...]
    logits = lax.dot_general(x, wg, (((1,), (1,)), ((), ())),
                             preferred_element_type=jnp.float32)
    logits = logits + bg_ref[...]
    m = jnp.max(logits, axis=1, keepdims=True)
    ex = jnp.exp(logits - m)
    p = ex / jnp.sum(ex, axis=1, keepdims=True)

    iota8 = lax.broadcasted_iota(jnp.int32, (T, NUM_EXPERTS), 1)
    p0 = jnp.max(p, axis=1, keepdims=True)
    e0 = jnp.min(jnp.where(p == p0, iota8, NUM_EXPERTS + 1), axis=1,
                 keepdims=True)
    pm = jnp.where(iota8 == e0, -jnp.inf, p)
    p1 = jnp.max(pm, axis=1, keepdims=True)
    e1 = jnp.min(jnp.where(pm == p1, iota8, NUM_EXPERTS + 1), axis=1,
                 keepdims=True)

    oh0 = (iota8 == e0).astype(jnp.float32)
    oh1 = (iota8 == e1).astype(jnp.float32)
    M = jnp.concatenate([oh0, oh1], axis=1)

    row_i = lax.broadcasted_iota(jnp.int32, (T, T), 0)
    col_j = lax.broadcasted_iota(jnp.int32, (T, T), 1)
    tri = (col_j < row_i).astype(jnp.bfloat16)
    pos = lax.dot_general(tri, M.astype(jnp.bfloat16),
                          (((1,), (0,)), ((), ())),
                          preferred_element_type=jnp.float32)

    counts = jnp.sum(M, axis=0, keepdims=True)
    kept = jnp.minimum(counts, float(CAP))
    kept0 = kept[:, :NUM_EXPERTS]
    kept1 = kept[:, NUM_EXPERTS:]
    n_e = kept0 + kept1
    nblk = jnp.ceil(n_e / float(BLK))

    tri8 = (lax.broadcasted_iota(jnp.int32, (NUM_EXPERTS, NUM_EXPERTS), 0) <
            lax.broadcasted_iota(jnp.int32, (NUM_EXPERTS, NUM_EXPERTS), 1))
    offb = lax.dot_general(nblk, tri8.astype(jnp.float32),
                           (((1,), (0,)), ((), ())),
                           preferred_element_type=jnp.float32)
    off = offb * float(BLK)
    cumb = offb + nblk

    pos0 = pos[:, :NUM_EXPERTS]
    pos1 = pos[:, NUM_EXPERTS:]
    pos_t0 = jnp.sum(oh0 * pos0, axis=1, keepdims=True)
    pos_t1 = jnp.sum(oh1 * pos1, axis=1, keepdims=True)
    off_t0 = jnp.sum(oh0 * off, axis=1, keepdims=True)
    off_t1 = jnp.sum(oh1 * off, axis=1, keepdims=True)
    kept0_t1 = jnp.sum(oh1 * kept0, axis=1, keepdims=True)

    keep0 = pos_t0 < float(CAP)
    keep1 = pos_t1 < float(CAP)
    dest0 = (off_t0 + pos_t0).astype(jnp.int32)
    dest1 = (off_t1 + kept0_t1 + pos_t1).astype(jnp.int32)

    tok = lax.broadcasted_iota(jnp.int32, (T, 1), 0)
    trash = NR + (tok & (NTRASH - 1))
    ds0_ref[...] = jnp.where(keep0, dest0, trash)
    ds1_ref[...] = jnp.where(keep1, dest1, trash)
    dg0_ref[...] = jnp.where(keep0, dest0, 0)
    dg1_ref[...] = jnp.where(keep1, dest1, 0)
    dh0_ref[...] = jnp.where(keep0, dest0, 0) + NR
    dh1_ref[...] = jnp.where(keep1, dest1, 0) + NR
    g0_ref[...] = jnp.broadcast_to(jnp.where(keep0, p0, 0.0), (T, 16))
    g1_ref[...] = jnp.broadcast_to(jnp.where(keep1, p1, 0.0), (T, 16))

    bid = lax.broadcasted_iota(jnp.int32, (NB, NUM_EXPERTS), 0)
    eb = jnp.sum((bid >= cumb.astype(jnp.int32)).astype(jnp.int32),
                 axis=1, keepdims=True)
    total_b = jnp.sum(nblk, axis=1, keepdims=True).astype(jnp.int32)
    be_ref[...] = jnp.minimum(eb, NUM_EXPERTS - 1)
    bv_ref[...] = (lax.broadcasted_iota(jnp.int32, (NB, 1), 0)
                   < total_b).astype(jnp.int32)


def _router(x, wg, bg):
    outs = pl.pallas_call(
        _router_body,
        out_shape=[
            jax.ShapeDtypeStruct((T, 1), jnp.int32),
            jax.ShapeDtypeStruct((T, 1), jnp.int32),
            jax.ShapeDtypeStruct((T, 1), jnp.int32),
            jax.ShapeDtypeStruct((T, 1), jnp.int32),
            jax.ShapeDtypeStruct((T, 1), jnp.int32),
            jax.ShapeDtypeStruct((T, 1), jnp.int32),
            jax.ShapeDtypeStruct((T, 16), jnp.float32),
            jax.ShapeDtypeStruct((T, 16), jnp.float32),
            jax.ShapeDtypeStruct((NB, 1), jnp.int32),
            jax.ShapeDtypeStruct((NB, 1), jnp.int32),
        ],
    )(x, wg, bg.reshape(1, NUM_EXPERTS))
    return outs


def _dispatch_body(x_hbm, d0_hbm, d1_hbm, xs_hbm, xv, i0v, i1v, s0, s1):
    wid = lax.axis_index("s") * NC + lax.axis_index("c")
    base = wid * TPW
    pltpu.sync_copy(d0_hbm.at[pl.ds(base, TPW)], i0v)
    pltpu.sync_copy(d1_hbm.at[pl.ds(base, TPW)], i1v)
    pltpu.sync_copy(x_hbm.at[pl.ds(base, TPW)], xv)
    cp0 = pltpu.async_copy(xv, xs_hbm.at[i0v], s0)
    cp1 = pltpu.async_copy(xv, xs_hbm.at[i1v], s1)
    cp0.wait()
    cp1.wait()


@functools.cache
def _make_dispatch():
    return functools.partial(
        pl.kernel,
        out_type=jax.ShapeDtypeStruct((NR + NTRASH, D_MODEL), jnp.float32),
        mesh=plsc.VectorSubcoreMesh(core_axis_name="c", subcore_axis_name="s"),
        scratch_types=[
            pltpu.VMEM((TPW, D_MODEL), jnp.float32),
            pltpu.VMEM((TPW,), jnp.int32),
            pltpu.VMEM((TPW,), jnp.int32),
            pltpu.SemaphoreType.DMA,
            pltpu.SemaphoreType.DMA,
        ],
    )(_dispatch_body)


def _dispatch(x, ds0, ds1):
    return _make_dispatch()(x, ds0, ds1)


FBLK = 2048
NFF = D_FF // FBLK


def _ffn_body(be_ref, bv_ref, xs_ref, w1_ref, b1_ref, w2_ref, b2_ref, y_ref):
    f = pl.program_id(0)
    b = pl.program_id(1)

    @pl.when(bv_ref[b] != 0)
    def _():
        xb = xs_ref[...]
        h = lax.dot_general(xb, w1_ref[0], (((1,), (1,)), ((), ())),
                            preferred_element_type=jnp.float32)
        h = jnp.maximum(h + b1_ref[0], 0.0)
        part = lax.dot_general(h, w2_ref[0], (((1,), (1,)), ((), ())),
                               preferred_element_type=jnp.float32)
        bias_scale = jnp.where(f == 0, 1.0, 0.0)
        y_ref[0] = part + b2_ref[0] * bias_scale


def _ffn(be, bv, xs, w1, b1, w2, b2):
    grid_spec = pltpu.PrefetchScalarGridSpec(
        num_scalar_prefetch=2,
        grid=(NFF, NB),
        in_specs=[
            pl.BlockSpec((BLK, D_MODEL), lambda f, b, be, bv: (b, 0)),
            pl.BlockSpec((1, FBLK, D_MODEL),
                         lambda f, b, be, bv: (be[b], f, 0)),
            pl.BlockSpec((1, 1, FBLK), lambda f, b, be, bv: (be[b], 0, f)),
            pl.BlockSpec((1, D_MODEL, FBLK),
                         lambda f, b, be, bv: (be[b], 0, f)),
            pl.BlockSpec((1, 1, D_MODEL),
                         lambda f, b, be, bv: (be[b], 0, 0)),
        ],
        out_specs=pl.BlockSpec((1, BLK, D_MODEL),
                               lambda f, b, be, bv: (f, b, 0)),
    )
    y3 = pl.pallas_call(
        _ffn_body,
        grid_spec=grid_spec,
        out_shape=jax.ShapeDtypeStruct((NFF, NR, D_MODEL), jnp.float32),
    )(be, bv, xs, w1,
      b1.reshape(NUM_EXPERTS, 1, D_FF),
      w2,
      b2.reshape(NUM_EXPERTS, 1, D_MODEL))
    return y3.reshape(NFF * NR, D_MODEL)


def _combine_body(y_hbm, i0_hbm, i1_hbm, ih0_hbm, ih1_hbm, g0_hbm, g1_hbm,
                  out_hbm, r0v, r1v, rh0v, rh1v, ov, i0v, i1v, ih0v, ih1v,
                  g0v, g1v, s0, s1, s2, s3):
    wid = lax.axis_index("s") * NC + lax.axis_index("c")
    for chunk in range(TPW // CHUNK):
        base = wid * TPW + chunk * CHUNK
        pltpu.sync_copy(i0_hbm.at[pl.ds(base, CHUNK)], i0v)
        pltpu.sync_copy(i1_hbm.at[pl.ds(base, CHUNK)], i1v)
        pltpu.sync_copy(ih0_hbm.at[pl.ds(base, CHUNK)], ih0v)
        pltpu.sync_copy(ih1_hbm.at[pl.ds(base, CHUNK)], ih1v)
        pltpu.sync_copy(g0_hbm.at[pl.ds(base, CHUNK)], g0v)
        pltpu.sync_copy(g1_hbm.at[pl.ds(base, CHUNK)], g1v)
        cp0 = pltpu.async_copy(y_hbm.at[i0v], r0v, s0)
        cp1 = pltpu.async_copy(y_hbm.at[i1v], r1v, s1)
        cp2 = pltpu.async_copy(y_hbm.at[ih0v], rh0v, s2)
        cp3 = pltpu.async_copy(y_hbm.at[ih1v], rh1v, s3)
        cp0.wait()
        cp1.wait()
        cp2.wait()
        cp3.wait()

        def body(i, _):
            g0s = g0v[i, :]
            g1s = g1v[i, :]
            for j in range(D_MODEL // 16):
                sl = pl.ds(j * 16, 16)
                ov[i, sl] = (g0s * (r0v[i, sl] + rh0v[i, sl])
                             + g1s * (r1v[i, sl] + rh1v[i, sl]))
            return 0

        lax.fori_loop(0, CHUNK, body, 0)
        pltpu.sync_copy(ov, out_hbm.at[pl.ds(base, CHUNK)])


@functools.cache
def _make_combine():
    return functools.partial(
        pl.kernel,
        out_type=jax.ShapeDtypeStruct((T, D_MODEL), jnp.float32),
        mesh=plsc.VectorSubcoreMesh(core_axis_name="c", subcore_axis_name="s"),
        scratch_types=[
            pltpu.VMEM((CHUNK, D_MODEL), jnp.float32),
            pltpu.VMEM((CHUNK, D_MODEL), jnp.float32),
            pltpu.VMEM((CHUNK, D_MODEL), jnp.float32),
            pltpu.VMEM((CHUNK, D_MODEL), jnp.float32),
            pltpu.VMEM((CHUNK, D_MODEL), jnp.float32),
            pltpu.VMEM((CHUNK,), jnp.int32),
            pltpu.VMEM((CHUNK,), jnp.int32),
            pltpu.VMEM((CHUNK,), jnp.int32),
            pltpu.VMEM((CHUNK,), jnp.int32),
            pltpu.VMEM((CHUNK, 16), jnp.float32),
            pltpu.VMEM((CHUNK, 16), jnp.float32),
            pltpu.SemaphoreType.DMA,
            pltpu.SemaphoreType.DMA,
            pltpu.SemaphoreType.DMA,
            pltpu.SemaphoreType.DMA,
        ],
    )(_combine_body)


def _combine(y, dg0, dg1, dh0, dh1, g0, g1):
    return _make_combine()(y, dg0, dg1, dh0, dh1, g0, g1)


def kernel(h, w1, b1, w2, b2, wg, bg):
    Bb, Ll, D = h.shape
    x = h.reshape(T, D_MODEL)
    ds0, ds1, dg0, dg1, dh0, dh1, g0, g1, be, bv = _router(x, wg, bg)
    xs = _dispatch(x, ds0.reshape(T), ds1.reshape(T))
    y = _ffn(be.reshape(NB), bv.reshape(NB), xs, w1, b1, w2, b2)
    out = _combine(y, dg0.reshape(T), dg1.reshape(T),
                   dh0.reshape(T), dh1.reshape(T), g0, g1)
    return out.reshape(Bb, Ll, D)
